# Initial kernel scaffold; baseline (speedup 1.0000x reference)
import jax
import jax.numpy as jnp
from jax.experimental import pallas as pl


def kernel(x, edge_index, W, b):
    raise NotImplementedError("write your pallas kernel here")


# speedup vs baseline: 4.2961x; 4.2961x over previous
import dataclasses
import functools

import jax
import jax.numpy as jnp
from jax import lax
from jax.experimental import pallas as pl
from jax.experimental.pallas import tpu as pltpu
from jax.experimental.pallas import tpu_sc as plsc

N = 10000
E = 160000
D = 256
NC = 2
NS = 16
NPAD = 10240
HALF = 5120
EPT = 10240
NB = 80
K = 128
PADE = 16 * EPT
DUMP = HALF
ACC_ROWS = 5248
ROWS_PT = 328

_mesh = plsc.VectorSubcoreMesh(core_axis_name="c", subcore_axis_name="s")

_sc_params = pltpu.CompilerParams()
if "needs_layout_passes" in pltpu.CompilerParams.__dataclass_fields__:
    _sc_params = dataclasses.replace(_sc_params, needs_layout_passes=False)


HW = 128


def _hist(loc_r, ones_v, zdeg):
    @functools.partial(
        pl.kernel,
        out_type=jax.ShapeDtypeStruct((NC, HALF, HW), jnp.float32),
        mesh=_mesh,
        scratch_types=[
            pltpu.VMEM((NB, K), jnp.int32),
            pltpu.VMEM((K, HW), jnp.float32),
            pltpu.VMEM_SHARED((ACC_ROWS, HW), jnp.float32),
        ],
        compiler_params=_sc_params,
    )
    def k(loc_hbm, ones_hbm, zdeg_hbm, deg_out, idx2d, ones_vm, deg_sh):
        c = lax.axis_index("c")
        s = lax.axis_index("s")
        pltpu.sync_copy(loc_hbm.at[c, s], idx2d)
        pltpu.sync_copy(ones_hbm, ones_vm)
        pltpu.sync_copy(zdeg_hbm, deg_sh.at[pl.ds(s * ROWS_PT, ROWS_PT)])
        plsc.subcore_barrier()

        @pl.loop(0, NB)
        def _(r):
            pltpu.sync_copy(ones_vm, deg_sh.at[idx2d.at[r]], add=True)

        plsc.subcore_barrier()
        pltpu.sync_copy(deg_sh.at[pl.ds(s * 320, 320)],
                        deg_out.at[c, pl.ds(s * 320, 320)])

    return k(loc_r, ones_v, zdeg)


def _matmul_scale(x, deg2d, W):
    bm = 400

    def body(x_ref, d_ref, w_ref, o0_ref, o1_ref):
        deg = d_ref[...]
        dis = jnp.where(deg > 0.5, lax.rsqrt(jnp.maximum(deg, 1e-12)), 0.0)
        y = jnp.dot(x_ref[...] * dis, w_ref[...],
                    preferred_element_type=jnp.float32)
        o0_ref[...] = y[:, :128]
        o1_ref[...] = y[:, 128:]

    return pl.pallas_call(
        body,
        grid=(N // bm,),
        in_specs=[
            pl.BlockSpec((bm, D), lambda i: (i, 0)),
            pl.BlockSpec((bm, 1), lambda i: (i, 0)),
            pl.BlockSpec((D, D), lambda i: (0, 0)),
        ],
        out_specs=[pl.BlockSpec((bm, 128), lambda i: (i, 0)),
                   pl.BlockSpec((bm, 128), lambda i: (i, 0))],
        out_shape=[jax.ShapeDtypeStruct((N, 128), jnp.float32),
                   jax.ShapeDtypeStruct((N, 128), jnp.float32)],
    )(x, deg2d, W)


def _aggregate(y0, y1, src_r, loc_r, zacc):
    DH = 128
    HNB = NB // 2
    HCAP = HNB * K

    @functools.partial(
        pl.kernel,
        out_type=jax.ShapeDtypeStruct((N, D), jnp.float32),
        mesh=_mesh,
        scratch_types=[
            pltpu.VMEM((NB, K), jnp.int32),
            pltpu.VMEM((NB, K), jnp.int32),
            pltpu.VMEM((NB, K), jnp.int32),
            pltpu.VMEM((NB, K), jnp.int32),
            pltpu.VMEM((HCAP,), jnp.int32),
            pltpu.VMEM((HCAP,), jnp.int32),
            [pltpu.VMEM((K, DH), jnp.float32) for _ in range(2)],
            [pltpu.SemaphoreType.DMA for _ in range(2)],
            [pltpu.SemaphoreType.DMA for _ in range(2)],
            pltpu.VMEM_SHARED((ACC_ROWS, DH), jnp.float32),
        ],
        compiler_params=_sc_params,
    )
    def k(y0_hbm, y1_hbm, src_hbm, loc_hbm, zacc_hbm, out_hbm, raws, rawd,
          src2d, dst2d, fsrc, fdst, bufs, gsems, ssems, acc_sh):
        c = lax.axis_index("c")
        s = lax.axis_index("s")
        pltpu.sync_copy(src_hbm.at[s], raws)
        pltpu.sync_copy(loc_hbm.at[c, s], rawd)

        dumpv = jnp.full((16,), DUMP, jnp.int32) + s
        zerov = jnp.zeros((16,), jnp.int32)
        nbs = []
        for p in range(2):
            @pl.loop(0, HNB)
            def _(r):
                for kk in range(K // 16):
                    o = r * K + kk * 16
                    fsrc[pl.ds(o, 16)] = zerov
                    fdst[pl.ds(o, 16)] = dumpv

            def cbody(r, cnt):
                for kk in range(K // 16):
                    dv = rawd[r, pl.ds(kk * 16, 16)]
                    sv = raws[r, pl.ds(kk * 16, 16)]
                    ok = dv < HALF
                    plsc.store_compressed(fsrc.at[pl.ds(cnt, 16)], sv,
                                          mask=ok)
                    plsc.store_compressed(fdst.at[pl.ds(cnt, 16)], dv,
                                          mask=ok)
                    cnt = cnt + jnp.sum(ok.astype(jnp.int32))
                return cnt

            cnt = lax.fori_loop(p * HNB, (p + 1) * HNB, cbody, jnp.int32(0))
            nb = lax.div(cnt + (2 * K - 1), jnp.int32(2 * K)) * 2
            nb = jnp.maximum(nb, 2)
            nbs.append(nb)

            @pl.loop(0, nb)
            def _(r):
                for kk in range(K // 16):
                    o = r * K + kk * 16
                    src2d[p * HNB + r, pl.ds(kk * 16, 16)] = fsrc[pl.ds(o, 16)]
                    dst2d[p * HNB + r, pl.ds(kk * 16, 16)] = fdst[pl.ds(o, 16)]

        for h in range(2):
            yh = y0_hbm if h == 0 else y1_hbm
            pltpu.sync_copy(zacc_hbm, acc_sh.at[pl.ds(s * ROWS_PT, ROWS_PT)])
            plsc.subcore_barrier()

            for p in range(2):
                base = p * HNB
                nb = nbs[p]
                for j in range(2):
                    pltpu.async_copy(yh.at[src2d.at[base + j]], bufs[j],
                                     gsems[j])

                @pl.loop(0, nb, step=2)
                def _(q):
                    for j in range(2):
                        pltpu.make_async_copy(yh.at[src2d.at[base + q + j]],
                                              bufs[j], gsems[j]).wait()
                        pltpu.async_copy(bufs[j],
                                         acc_sh.at[dst2d.at[base + q + j]],
                                         ssems[j], add=True)
                    for j in range(2):
                        pltpu.make_async_copy(bufs[j],
                                              acc_sh.at[dst2d.at[base + q + j]],
                                              ssems[j]).wait()

                        @pl.when(q + 2 + j < nb)
                        def _():
                            pltpu.async_copy(
                                yh.at[src2d.at[base + q + 2 + j]], bufs[j],
                                gsems[j])

            plsc.subcore_barrier()

            @pl.when(c == 0)
            def _():
                pltpu.sync_copy(
                    acc_sh.at[pl.ds(s * 320, 320)],
                    out_hbm.at[pl.ds(s * 320, 320), pl.ds(h * DH, DH)])

            @pl.when((c == 1) & (s < 15))
            def _():
                pltpu.sync_copy(
                    acc_sh.at[pl.ds(s * 304, 304)],
                    out_hbm.at[pl.ds(HALF + s * 304, 304), pl.ds(h * DH, DH)])

            @pl.when((c == 1) & (s == 15))
            def _():
                pltpu.sync_copy(
                    acc_sh.at[pl.ds(4560, 320)],
                    out_hbm.at[pl.ds(HALF + 4560, 320), pl.ds(h * DH, DH)])

            plsc.subcore_barrier()

    return k(y0, y1, src_r, loc_r, zacc)


def _bias_relu_scale(pre, deg2d, b2d):
    bm = 400

    def body(a_ref, d_ref, b_ref, o_ref):
        deg = d_ref[...]
        dis = jnp.where(deg > 0.5, lax.rsqrt(jnp.maximum(deg, 1e-12)), 0.0)
        o_ref[...] = jnp.maximum(a_ref[...] * dis + b_ref[...], 0.0)

    return pl.pallas_call(
        body,
        grid=(N // bm,),
        in_specs=[
            pl.BlockSpec((bm, D), lambda i: (i, 0)),
            pl.BlockSpec((bm, 1), lambda i: (i, 0)),
            pl.BlockSpec((1, D), lambda i: (0, 0)),
        ],
        out_specs=pl.BlockSpec((bm, D), lambda i: (i, 0)),
        out_shape=jax.ShapeDtypeStruct((N, D), jnp.float32),
    )(pre, deg2d, b2d)


def kernel(x, edge_index, W, b):
    src = edge_index[0]
    dst = edge_index[1]
    pad = PADE - E
    src_r = jnp.concatenate([src, jnp.zeros((pad,), jnp.int32)]).reshape(
        16, NB, K)
    dstp = jnp.concatenate([dst, jnp.full((pad,), NPAD, jnp.int32)])
    dump = DUMP + jnp.repeat(jnp.arange(16, dtype=jnp.int32), EPT)
    loc_list = []
    for core in range(NC):
        lc = dstp - core * HALF
        lc = jnp.where((lc >= 0) & (lc < HALF), lc, dump)
        loc_list.append(lc.reshape(16, NB, K))
    loc_r = jnp.stack(loc_list)
    ones_v = jnp.ones((K, HW), jnp.float32)
    zdeg = jnp.zeros((ROWS_PT, HW), jnp.float32)
    zacc = jnp.zeros((ROWS_PT, 128), jnp.float32)

    deg = _hist(loc_r, ones_v, zdeg)
    deg2d = jnp.concatenate([deg[0, :, 0], deg[1, :, 0]])[:N].reshape(N, 1)
    y0, y1 = _matmul_scale(x, deg2d, W)
    pre = _aggregate(y0, y1, src_r, loc_r, zacc)
    return _bias_relu_scale(pre, deg2d, b.reshape(1, D))
